# Initial kernel scaffold; baseline (speedup 1.0000x reference)
import jax
import jax.numpy as jnp
from jax.experimental import pallas as pl


def kernel(hidden_states, topk_weights, topk_ids, gate_up_weight, down_weight):
    raise NotImplementedError("write your pallas kernel here")


# speedup vs baseline: 1.9199x; 1.9199x over previous
import functools

import jax
import jax.numpy as jnp
from jax.experimental import pallas as pl


def _moe_kernel(x_ref, g_ref, u_ref, d_ref, tw_ref, ids_ref, o_ref):
    e = pl.program_id(0)
    f = pl.program_id(1)

    @pl.when((e == 0) & (f == 0))
    def _init():
        o_ref[...] = jnp.zeros_like(o_ref)

    x = x_ref[...]
    g = g_ref[0]
    u = u_ref[0]
    d = d_ref[0]

    gate = jax.lax.dot_general(x, g, (((1,), (1,)), ((), ())),
                               preferred_element_type=jnp.float32)
    up = jax.lax.dot_general(x, u, (((1,), (1,)), ((), ())),
                             preferred_element_type=jnp.float32)
    act = (gate * jax.lax.logistic(gate)) * up
    eo = jax.lax.dot_general(act, d, (((1,), (1,)), ((), ())),
                             preferred_element_type=jnp.float32)

    ids = ids_ref[...]
    tw = tw_ref[...]
    w = jnp.sum(jnp.where(ids == e, tw, 0.0), axis=1)
    o_ref[...] += w[:, None] * eo


@functools.partial(jax.jit, static_argnames=())
def kernel(hidden_states, topk_weights, topk_ids, gate_up_weight, down_weight):
    T, D = hidden_states.shape
    E, two_ffn, _ = gate_up_weight.shape
    ffn = two_ffn // 2
    F = 1024
    nf = ffn // F

    grid = (E, nf)
    out = pl.pallas_call(
        _moe_kernel,
        grid=grid,
        in_specs=[
            pl.BlockSpec((T, D), lambda e, f: (0, 0)),
            pl.BlockSpec((1, F, D), lambda e, f: (e, f, 0)),
            pl.BlockSpec((1, F, D), lambda e, f, _nf=nf: (e, f + _nf, 0)),
            pl.BlockSpec((1, D, F), lambda e, f: (e, 0, f)),
            pl.BlockSpec(topk_weights.shape, lambda e, f: (0, 0)),
            pl.BlockSpec(topk_ids.shape, lambda e, f: (0, 0)),
        ],
        out_specs=pl.BlockSpec((T, D), lambda e, f: (0, 0)),
        out_shape=jax.ShapeDtypeStruct((T, D), jnp.float32),
    )(hidden_states, gate_up_weight, gate_up_weight, down_weight,
      topk_weights, topk_ids)
    return out
